# Initial kernel scaffold; baseline (speedup 1.0000x reference)
import jax
import jax.numpy as jnp
from jax.experimental import pallas as pl


def kernel(features, neigh_idx):
    raise NotImplementedError("write your pallas kernel here")


# speedup vs baseline: 7.7713x; 7.7713x over previous
import functools

import jax
import jax.numpy as jnp
from jax import lax
from jax.experimental import pallas as pl
from jax.experimental.pallas import tpu as pltpu
from jax.experimental.pallas import tpu_sc as plsc

N_CORES = 2
N_SUBCORES = 16
NW = N_CORES * N_SUBCORES
C = 112
S = 10
D = 128
LANES = 16


def _body(feat_hbm, idx3_hbm, out_hbm, idx_v, acc_v, out_v, sem, *, K):
    wid = lax.axis_index("s") * N_CORES + lax.axis_index("c")
    zeros = jnp.zeros((LANES,), jnp.float32)
    inv = jnp.full((LANES,), 1.0 / S, jnp.float32)

    @pl.loop(0, C)
    def _zero(i):
        for j in range(D // LANES):
            acc_v[i, pl.ds(j * LANES, LANES)] = zeros

    @pl.loop(0, K)
    def _chunk(g):
        chunk = wid * K + g
        pltpu.sync_copy(idx3_hbm.at[chunk], idx_v)
        cps = [
            pltpu.async_copy(feat_hbm.at[idx_v.at[si]], acc_v, sem, add=True)
            for si in range(S)
        ]
        for cp in cps:
            cp.wait()

        @pl.loop(0, C)
        def _scale(i):
            for j in range(D // LANES):
                sl = pl.ds(j * LANES, LANES)
                out_v[i, sl] = acc_v[i, sl] * inv
                acc_v[i, sl] = zeros

        pltpu.sync_copy(out_v, out_hbm.at[pl.ds(chunk * C, C)])


@functools.partial(jax.jit, static_argnames=("b_pad", "k_chunks"))
def _gather_mean(features, idx3, *, b_pad, k_chunks):
    mesh = plsc.VectorSubcoreMesh(core_axis_name="c", subcore_axis_name="s")
    kfn = pl.kernel(
        functools.partial(_body, K=k_chunks),
        out_type=jax.ShapeDtypeStruct((b_pad, D), jnp.float32),
        mesh=mesh,
        scratch_types=[
            pltpu.VMEM((S, C), jnp.int32),
            pltpu.VMEM((C, D), jnp.float32),
            pltpu.VMEM((C, D), jnp.float32),
            pltpu.SemaphoreType.DMA,
        ],
    )
    return kfn(features, idx3)


def kernel(features, neigh_idx):
    b = neigh_idx.shape[0]
    k_chunks = -(-b // (NW * C))
    b_pad = NW * C * k_chunks
    idx = neigh_idx.astype(jnp.int32)
    idx = jnp.pad(idx, ((0, b_pad - b), (0, 0)))
    idx3 = idx.reshape(b_pad // C, C, S).transpose(0, 2, 1)
    out = _gather_mean(features, idx3, b_pad=b_pad, k_chunks=k_chunks)
    return out[:b]


# speedup vs baseline: 8.9505x; 1.1517x over previous
import functools

import jax
import jax.numpy as jnp
from jax import lax
from jax.experimental import pallas as pl
from jax.experimental.pallas import tpu as pltpu
from jax.experimental.pallas import tpu_sc as plsc

N_CORES = 2
N_SUBCORES = 16
NW = N_CORES * N_SUBCORES
C = 112
S = 10
D = 128
LANES = 16


def _body(feat_hbm, idx3_hbm, out_hbm, idx_v, acc_v, out_v, isem, gsem, osem,
          *, K):
    wid = lax.axis_index("s") * N_CORES + lax.axis_index("c")
    base = wid * K
    zeros = jnp.zeros((LANES,), jnp.float32)
    inv = jnp.full((LANES,), 1.0 / S, jnp.float32)

    @pl.loop(0, C)
    def _zero(i):
        for j in range(D // LANES):
            sl = pl.ds(j * LANES, LANES)
            acc_v[0, i, sl] = zeros
            acc_v[1, i, sl] = zeros

    idx_d = [None] * K
    gat_d = [None] * K
    out_d = [None] * K

    def load_idx(g):
        idx_d[g] = pltpu.async_copy(idx3_hbm.at[base + g], idx_v.at[g % 2],
                                    isem)

    def fire_gathers(g):
        idx_d[g].wait()
        gat_d[g] = [
            pltpu.async_copy(feat_hbm.at[idx_v.at[g % 2].at[si]],
                             acc_v.at[g % 2], gsem, add=True)
            for si in range(S)
        ]

    def compute_store(g):
        p = g % 2
        if g >= 2:
            out_d[g - 2].wait()

        @pl.loop(0, C)
        def _scale(i):
            for j in range(D // LANES):
                sl = pl.ds(j * LANES, LANES)
                out_v[p, i, sl] = acc_v[p, i, sl] * inv
                acc_v[p, i, sl] = zeros

        out_d[g] = pltpu.async_copy(
            out_v.at[p], out_hbm.at[pl.ds((base + g) * C, C)], osem)

    load_idx(0)
    fire_gathers(0)
    if K > 1:
        load_idx(1)
    for g in range(K):
        if g + 1 < K:
            fire_gathers(g + 1)
        for cp in gat_d[g]:
            cp.wait()
        if g + 2 < K:
            load_idx(g + 2)
        compute_store(g)
    if K >= 2:
        out_d[K - 2].wait()
    out_d[K - 1].wait()


@functools.partial(jax.jit, static_argnames=("b_pad", "k_chunks"))
def _gather_mean(features, idx3, *, b_pad, k_chunks):
    mesh = plsc.VectorSubcoreMesh(core_axis_name="c", subcore_axis_name="s")
    kfn = pl.kernel(
        functools.partial(_body, K=k_chunks),
        out_type=jax.ShapeDtypeStruct((b_pad, D), jnp.float32),
        mesh=mesh,
        scratch_types=[
            pltpu.VMEM((2, S, C), jnp.int32),
            pltpu.VMEM((2, C, D), jnp.float32),
            pltpu.VMEM((2, C, D), jnp.float32),
            pltpu.SemaphoreType.DMA,
            pltpu.SemaphoreType.DMA,
            pltpu.SemaphoreType.DMA,
        ],
    )
    return kfn(features, idx3)


def kernel(features, neigh_idx):
    b = neigh_idx.shape[0]
    k_chunks = -(-b // (NW * C))
    b_pad = NW * C * k_chunks
    idx = neigh_idx.astype(jnp.int32)
    idx = jnp.pad(idx, ((0, b_pad - b), (0, 0)))
    idx3 = idx.reshape(b_pad // C, C, S).transpose(0, 2, 1)
    out = _gather_mean(features, idx3, b_pad=b_pad, k_chunks=k_chunks)
    return out[:b]
